# baseline (speedup 1.0000x reference)
import jax
import jax.numpy as jnp
import numpy as np
from jax.experimental import pallas as pl
from jax.experimental.pallas import tpu as pltpu

N = 32768
NUM_THETAS = 32
BUMP_STEPS = 32
R = 1.1
NUM_SEGMENTS = 16
BLK = 32768
KCHUNK = 2048

_LIN = np.linspace(-R, R, BUMP_STEPS, dtype=np.float32)
_LINP = np.repeat(100.0 * _LIN, NUM_THETAS).reshape(1, BUMP_STEPS * NUM_THETAS)


def _ect_kernel(x_ref, batch_ref, v_ref, lhi_ref, llo_ref, out_ref, acc_ref):
    i = pl.program_id(0)

    @pl.when(i == 0)
    def _():
        acc_ref[...] = jnp.zeros_like(acc_ref)

    xs = x_ref[...] * (-100.0)
    nhn = jax.lax.dot_general(
        xs, v_ref[...], (((1,), (1,)), ((), ())),
        preferred_element_type=jnp.float32).astype(jnp.bfloat16)

    seg = batch_ref[...]
    row = jax.lax.broadcasted_iota(jnp.int32, (NUM_SEGMENTS, BLK), 0)
    oh = (seg == row).astype(jnp.float32)
    cnt = jnp.sum(oh, axis=1, keepdims=True)
    ohb = oh.astype(jnp.bfloat16)

    acc = None
    for k in range(BLK // KCHUNK):
        sl = slice(k * KCHUNK, (k + 1) * KCHUNK)
        y = (jnp.tile(nhn[sl, :], (1, BUMP_STEPS)) + lhi_ref[...]) + llo_ref[...]
        th = jnp.tanh(y)
        d = jnp.dot(ohb[:, sl], th, preferred_element_type=jnp.float32)
        acc = d if acc is None else acc + d
    acc_ref[...] += 0.5 * acc + 0.5 * cnt

    @pl.when(i == (N // BLK) - 1)
    def _():
        out_ref[...] = acc_ref[...].reshape(
            NUM_SEGMENTS, BUMP_STEPS, NUM_THETAS)


@jax.jit
def kernel(x, batch, v):
    st = BUMP_STEPS * NUM_THETAS
    linp = jnp.asarray(_LINP)
    lhi = linp.astype(jnp.bfloat16)
    llo = (linp - lhi.astype(jnp.float32)).astype(jnp.bfloat16)
    return pl.pallas_call(
        _ect_kernel,
        grid=(N // BLK,),
        in_specs=[
            pl.BlockSpec((BLK, 2), lambda i: (i, 0)),
            pl.BlockSpec((1, BLK), lambda i: (0, i)),
            pl.BlockSpec((NUM_THETAS, 2), lambda i: (0, 0)),
            pl.BlockSpec((1, st), lambda i: (0, 0)),
            pl.BlockSpec((1, st), lambda i: (0, 0)),
        ],
        out_specs=pl.BlockSpec(
            (NUM_SEGMENTS, BUMP_STEPS, NUM_THETAS), lambda i: (0, 0, 0)),
        out_shape=jax.ShapeDtypeStruct(
            (NUM_SEGMENTS, BUMP_STEPS, NUM_THETAS), jnp.float32),
        scratch_shapes=[pltpu.VMEM((NUM_SEGMENTS, st), jnp.float32)],
    )(x, batch.reshape(1, N), v, lhi, llo)


# speedup vs baseline: 1.1034x; 1.1034x over previous
import jax
import jax.numpy as jnp
import numpy as np
from jax.experimental import pallas as pl
from jax.experimental.pallas import tpu as pltpu

N = 32768
NUM_THETAS = 32
BUMP_STEPS = 32
R = 1.1
NUM_SEGMENTS = 16
BLK = 8192
KCHUNK = 2048

_LIN = np.linspace(-R, R, BUMP_STEPS, dtype=np.float32)
_LINP = np.repeat(100.0 * _LIN, NUM_THETAS).reshape(1, BUMP_STEPS * NUM_THETAS)


def _ect_kernel(x_ref, batch_ref, v_ref, lhi_ref, llo_ref, out_ref, acc_ref):
    i = pl.program_id(0)

    @pl.when(i == 0)
    def _():
        acc_ref[...] = jnp.zeros_like(acc_ref)

    xs = x_ref[...] * (-100.0)
    nhn = jax.lax.dot_general(
        xs, v_ref[...], (((1,), (1,)), ((), ())),
        preferred_element_type=jnp.float32).astype(jnp.bfloat16)

    seg = batch_ref[...]
    row = jax.lax.broadcasted_iota(jnp.int32, (NUM_SEGMENTS, BLK), 0)
    oh = (seg == row).astype(jnp.float32)
    cnt = jnp.sum(oh, axis=1, keepdims=True)
    ohb = oh.astype(jnp.bfloat16)

    acc = None
    for k in range(BLK // KCHUNK):
        sl = slice(k * KCHUNK, (k + 1) * KCHUNK)
        y = (jnp.tile(nhn[sl, :], (1, BUMP_STEPS)) + lhi_ref[...]) + llo_ref[...]
        th = jnp.tanh(y)
        d = jnp.dot(ohb[:, sl], th, preferred_element_type=jnp.float32)
        acc = d if acc is None else acc + d
    acc_ref[...] += 0.5 * acc + 0.5 * cnt

    @pl.when(i == (N // BLK) - 1)
    def _():
        out_ref[...] = acc_ref[...].reshape(
            NUM_SEGMENTS, BUMP_STEPS, NUM_THETAS)


@jax.jit
def kernel(x, batch, v):
    st = BUMP_STEPS * NUM_THETAS
    linp = jnp.asarray(_LINP)
    lhi = linp.astype(jnp.bfloat16)
    llo = (linp - lhi.astype(jnp.float32)).astype(jnp.bfloat16)
    return pl.pallas_call(
        _ect_kernel,
        grid=(N // BLK,),
        in_specs=[
            pl.BlockSpec((BLK, 2), lambda i: (i, 0)),
            pl.BlockSpec((1, BLK), lambda i: (0, i)),
            pl.BlockSpec((NUM_THETAS, 2), lambda i: (0, 0)),
            pl.BlockSpec((1, st), lambda i: (0, 0)),
            pl.BlockSpec((1, st), lambda i: (0, 0)),
        ],
        out_specs=pl.BlockSpec(
            (NUM_SEGMENTS, BUMP_STEPS, NUM_THETAS), lambda i: (0, 0, 0)),
        out_shape=jax.ShapeDtypeStruct(
            (NUM_SEGMENTS, BUMP_STEPS, NUM_THETAS), jnp.float32),
        scratch_shapes=[pltpu.VMEM((NUM_SEGMENTS, st), jnp.float32)],
    )(x, batch.reshape(1, N), v, lhi, llo)
